# baseline (speedup 1.0000x reference)
import jax
import jax.numpy as jnp
from jax.experimental import pallas as pl
from jax.experimental.pallas import tpu as pltpu
from jax.experimental.pallas import tpu_sc as plsc


GATHER_WINDOW = 128


def _sc_gather_pairs(table2, pair_idx):
    b = pair_idx.shape[0]
    pair_idx = pair_idx.reshape(1, b)
    mesh = plsc.VectorSubcoreMesh(core_axis_name="core", subcore_axis_name="subcore")

    @pl.kernel(
        out_type=jax.ShapeDtypeStruct((b, table2.shape[1]), table2.dtype),
        mesh=mesh,
    )
    def kern(x_hbm, i_hbm, o_hbm):
        def body(i_vmem, o_vmem):
            pltpu.sync_copy(x_hbm.at[i_vmem.at[0]], o_vmem)

        pltpu.emit_pipeline(
            body,
            grid=(b // GATHER_WINDOW,),
            in_specs=[pl.BlockSpec((1, GATHER_WINDOW), index_map=lambda i: (0, i))],
            out_specs=[
                pl.BlockSpec(
                    (GATHER_WINDOW, table2.shape[1]), index_map=lambda i: (i, 0)
                )
            ],
            core_axis_name=("core", "subcore"),
            dimension_semantics=(pltpu.PARALLEL,),
        )(i_hbm, o_hbm)

    return kern(table2, pair_idx)


def _expand_body(par_ref, pair_ref, ft_ref, o_ref, emb_scratch):
    pair = pair_ref[...]
    par = par_ref[...]
    d = pair.shape[1] // 2
    emb_scratch[...] = jnp.where(par > 0, pair[:, d:], pair[:, :d])
    emb = emb_scratch[...]
    ft = ft_ref[...]
    t = ft.shape[1] // 4
    pieces = []
    for tt in range(t):
        pieces.append(emb)
        pieces.append(ft[:, 4 * tt : 4 * tt + 4])
    o_ref[...] = jnp.concatenate(pieces, axis=1)


def _tc_expand(parity, emb_pairs, ft, block_b=256):
    b, t, f = ft.shape
    d = emb_pairs.shape[1] // 2
    ft2 = ft.reshape(b, t * f)
    out_flat = pl.pallas_call(
        _expand_body,
        grid=(b // block_b,),
        in_specs=[
            pl.BlockSpec((block_b, 1), lambda i: (i, 0)),
            pl.BlockSpec((block_b, 2 * d), lambda i: (i, 0)),
            pl.BlockSpec((block_b, t * f), lambda i: (i, 0)),
        ],
        out_specs=pl.BlockSpec((block_b, t * (d + f)), lambda i: (i, 0)),
        out_shape=jax.ShapeDtypeStruct((b, t * (d + f)), jnp.float32),
        scratch_shapes=[pltpu.VMEM((block_b, d), jnp.float32)],
    )(parity, emb_pairs, ft2)
    return out_flat.reshape(b, t, d + f)


def kernel(feat_static, n_timesteps, feat_time, embedding_weight):
    idx = jnp.squeeze(feat_static.astype(jnp.int32), axis=-1)
    n, d = embedding_weight.shape
    table2 = embedding_weight.reshape(n // 2, 2 * d)
    emb_pairs = _sc_gather_pairs(table2, idx >> 1)
    parity = (idx & 1).reshape(-1, 1)
    return _tc_expand(parity, emb_pairs, feat_time)
